# baseline (speedup 1.0000x reference)
import functools

import jax
import jax.numpy as jnp
from jax import lax
from jax.experimental import pallas as pl
from jax.experimental.pallas import tpu as pltpu
from jax.experimental.pallas import tpu_sc as plsc

_INPUT_DIM = 8192
_OUTPUT_DIM = 64

_NC = 2
_NS = 16
_NW = _NC * _NS

_B = 4096 * 200
_BPW = _B // _NW
_CHUNK = 128
_NCHUNK = _BPW // _CHUNK
_NB = 8
_NGROUP = _NCHUNK // _NB

_mesh = plsc.VectorSubcoreMesh(core_axis_name="c", subcore_axis_name="s")


@functools.partial(
    pl.kernel,
    out_type=jax.ShapeDtypeStruct((_B // 2, 2 * _OUTPUT_DIM), jnp.float32),
    mesh=_mesh,
    scratch_types=[
        pltpu.VMEM((_NCHUNK, 2, _CHUNK // 2), jnp.int32),
        pltpu.VMEM((2, _NB, _CHUNK // 2, _OUTPUT_DIM), jnp.float32),
        [pltpu.SemaphoreType.DMA] * _NB,
        [pltpu.SemaphoreType.DMA] * _NB,
    ],
    compiler_params=pltpu.CompilerParams(use_tc_tiling_on_sc=False),
)
def _gather_kernel(idx_hbm, table_hbm, out_hbm, idx_v, rows_v, gsems, ssems):
    wid = lax.axis_index("s") * _NC + lax.axis_index("c")
    base2 = wid * (_BPW // 2)
    h = _CHUNK // 2

    pltpu.sync_copy(idx_hbm.at[wid], idx_v)

    def fire(j, b):
        pltpu.async_copy(table_hbm.at[idx_v.at[j, 0]], rows_v.at[0, b], gsems[b])
        pltpu.async_copy(table_hbm.at[idx_v.at[j, 1]], rows_v.at[1, b], gsems[b])

    for b in range(_NB):
        fire(b, b)

    def group(g, carry):
        j0 = g * _NB
        for b in range(_NB):
            for e in range(2):
                pltpu.make_async_copy(
                    table_hbm.at[idx_v.at[0, 0]], rows_v.at[e, b], gsems[b]
                ).wait()
            off = base2 + (j0 + b) * h
            for e in range(2):
                pltpu.async_copy(
                    rows_v.at[e, b],
                    out_hbm.at[pl.ds(off, h), pl.ds(e * _OUTPUT_DIM, _OUTPUT_DIM)],
                    ssems[b],
                )
        for b in range(_NB):
            for e in range(2):
                pltpu.make_async_copy(
                    rows_v.at[e, b],
                    out_hbm.at[pl.ds(base2, h), pl.ds(e * _OUTPUT_DIM, _OUTPUT_DIM)],
                    ssems[b],
                ).wait()

            @pl.when(g < _NGROUP - 1)
            def _():
                fire(j0 + _NB + b, b)

        return carry

    lax.fori_loop(0, _NGROUP, group, 0)


_D0_PER_BLK = 8


def _expand_body(in_ref, out_ref):
    x = in_ref[...]
    out_ref[0] = x[:, : _OUTPUT_DIM].reshape(_D0_PER_BLK, 200, _OUTPUT_DIM)
    out_ref[1] = x[:, _OUTPUT_DIM :].reshape(_D0_PER_BLK, 200, _OUTPUT_DIM)


_expand = pl.pallas_call(
    _expand_body,
    grid=(2048 // _D0_PER_BLK,),
    in_specs=[
        pl.BlockSpec((_D0_PER_BLK * 200, 128), lambda i: (i, 0)),
    ],
    out_specs=pl.BlockSpec(
        (2, _D0_PER_BLK, 200, _OUTPUT_DIM), lambda i: (0, i, 0, 0)
    ),
    out_shape=jax.ShapeDtypeStruct((2, 2048, 200, _OUTPUT_DIM), jnp.float32),
)


def kernel(inputs, pos_matrix):
    idx = inputs.reshape(2, _NW, _NCHUNK, _CHUNK // 2).transpose(1, 2, 0, 3)
    table = pos_matrix.reshape(_INPUT_DIM, -1)[:, :_OUTPUT_DIM]
    packed = _gather_kernel(idx, table)
    return _expand(packed).reshape(4096, 200, _OUTPUT_DIM)


# speedup vs baseline: 1.0379x; 1.0379x over previous
import functools

import jax
import jax.numpy as jnp
from jax import lax
from jax.experimental import pallas as pl
from jax.experimental.pallas import tpu as pltpu
from jax.experimental.pallas import tpu_sc as plsc

_INPUT_DIM = 8192
_OUTPUT_DIM = 64

_NC = 2
_NS = 16
_NW = _NC * _NS

_B = 4096 * 200
_BPW = _B // _NW
_CHUNK = 128
_NCHUNK = _BPW // _CHUNK
_NB = 8
_NGROUP = _NCHUNK // _NB

_mesh = plsc.VectorSubcoreMesh(core_axis_name="c", subcore_axis_name="s")


@functools.partial(
    pl.kernel,
    out_type=jax.ShapeDtypeStruct((_B // 2, 2 * _OUTPUT_DIM), jnp.float32),
    mesh=_mesh,
    scratch_types=[
        pltpu.VMEM((2, _NCHUNK, _CHUNK // 2), jnp.int32),
        pltpu.VMEM((2, _NB, _CHUNK // 2, _OUTPUT_DIM), jnp.float32),
        [pltpu.SemaphoreType.DMA] * _NB,
        [pltpu.SemaphoreType.DMA] * _NB,
    ],
    compiler_params=pltpu.CompilerParams(use_tc_tiling_on_sc=False),
)
def _gather_kernel(idx_hbm, table_hbm, out_hbm, idx_v, rows_v, gsems, ssems):
    wid = lax.axis_index("s") * _NC + lax.axis_index("c")
    base2 = wid * (_BPW // 2)
    h = _CHUNK // 2

    pltpu.sync_copy(idx_hbm.at[0, wid], idx_v.at[0])
    pltpu.sync_copy(idx_hbm.at[1, wid], idx_v.at[1])

    def fire(j, b):
        pltpu.async_copy(table_hbm.at[idx_v.at[0, j]], rows_v.at[0, b], gsems[b])
        pltpu.async_copy(table_hbm.at[idx_v.at[1, j]], rows_v.at[1, b], gsems[b])

    for b in range(_NB):
        fire(b, b)

    def group(g, carry):
        j0 = g * _NB
        for b in range(_NB):
            for e in range(2):
                pltpu.make_async_copy(
                    table_hbm.at[idx_v.at[0, 0]], rows_v.at[e, b], gsems[b]
                ).wait()
            off = base2 + (j0 + b) * h
            for e in range(2):
                pltpu.async_copy(
                    rows_v.at[e, b],
                    out_hbm.at[pl.ds(off, h), pl.ds(e * _OUTPUT_DIM, _OUTPUT_DIM)],
                    ssems[b],
                )
        for b in range(_NB):
            for e in range(2):
                pltpu.make_async_copy(
                    rows_v.at[e, b],
                    out_hbm.at[pl.ds(base2, h), pl.ds(e * _OUTPUT_DIM, _OUTPUT_DIM)],
                    ssems[b],
                ).wait()

            @pl.when(g < _NGROUP - 1)
            def _():
                fire(j0 + _NB + b, b)

        return carry

    lax.fori_loop(0, _NGROUP, group, 0)


_D0_PER_BLK = 8


def _expand_body(in_ref, out_ref):
    x = in_ref[...]
    out_ref[0] = x[:, : _OUTPUT_DIM].reshape(_D0_PER_BLK, 200, _OUTPUT_DIM)
    out_ref[1] = x[:, _OUTPUT_DIM :].reshape(_D0_PER_BLK, 200, _OUTPUT_DIM)


_expand = pl.pallas_call(
    _expand_body,
    grid=(2048 // _D0_PER_BLK,),
    in_specs=[
        pl.BlockSpec((_D0_PER_BLK * 200, 128), lambda i: (i, 0)),
    ],
    out_specs=pl.BlockSpec(
        (2, _D0_PER_BLK, 200, _OUTPUT_DIM), lambda i: (0, i, 0, 0)
    ),
    out_shape=jax.ShapeDtypeStruct((2, 2048, 200, _OUTPUT_DIM), jnp.float32),
)


def kernel(inputs, pos_matrix):
    idx = inputs.reshape(2, _NW, _NCHUNK, _CHUNK // 2)
    table = pos_matrix.reshape(_INPUT_DIM, -1)[:, :_OUTPUT_DIM]
    packed = _gather_kernel(idx, table)
    return _expand(packed).reshape(4096, 200, _OUTPUT_DIM)


# speedup vs baseline: 1.4417x; 1.3891x over previous
import functools

import jax
import jax.numpy as jnp
from jax import lax
from jax.experimental import pallas as pl
from jax.experimental.pallas import tpu as pltpu
from jax.experimental.pallas import tpu_sc as plsc

_INPUT_DIM = 8192
_OUTPUT_DIM = 64

_NC = 2
_NS = 16
_NW = _NC * _NS

_D0 = 4096
_P = 200
_B = _D0 * _P
_BPW = _B // _NW
_CHUNK = 128
_NCHUNK = _BPW // _CHUNK
_NB = 8
_NGROUP = _NCHUNK // _NB

_mesh = plsc.VectorSubcoreMesh(core_axis_name="c", subcore_axis_name="s")


@functools.partial(
    pl.kernel,
    out_type=jax.ShapeDtypeStruct((_B // 2, 2 * _OUTPUT_DIM), jnp.float32),
    mesh=_mesh,
    scratch_types=[
        pltpu.VMEM((_NCHUNK, 2, _CHUNK // 2), jnp.int32),
        pltpu.VMEM((2, _NB, _CHUNK // 2, _OUTPUT_DIM), jnp.float32),
        [pltpu.SemaphoreType.DMA] * _NB,
        [pltpu.SemaphoreType.DMA] * _NB,
    ],
    compiler_params=pltpu.CompilerParams(use_tc_tiling_on_sc=False),
)
def _gather_kernel(idx_hbm, table_hbm, out_hbm, idx_v, rows_v, gsems, ssems):
    wid = lax.axis_index("s") * _NC + lax.axis_index("c")
    base2 = wid * (_BPW // 2)
    h = _CHUNK // 2

    pltpu.sync_copy(idx_hbm.at[wid], idx_v)

    def fire(j, b):
        pltpu.async_copy(table_hbm.at[idx_v.at[j, 0]], rows_v.at[0, b], gsems[b])
        pltpu.async_copy(table_hbm.at[idx_v.at[j, 1]], rows_v.at[1, b], gsems[b])

    for b in range(_NB):
        fire(b, b)

    def group(g, carry):
        j0 = g * _NB
        for b in range(_NB):
            for e in range(2):
                pltpu.make_async_copy(
                    table_hbm.at[idx_v.at[0, 0]], rows_v.at[e, b], gsems[b]
                ).wait()
            off = base2 + (j0 + b) * h
            for e in range(2):
                pltpu.async_copy(
                    rows_v.at[e, b],
                    out_hbm.at[pl.ds(off, h), pl.ds(e * _OUTPUT_DIM, _OUTPUT_DIM)],
                    ssems[b],
                )
        for b in range(_NB):
            for e in range(2):
                pltpu.make_async_copy(
                    rows_v.at[e, b],
                    out_hbm.at[pl.ds(base2, h), pl.ds(e * _OUTPUT_DIM, _OUTPUT_DIM)],
                    ssems[b],
                ).wait()

            @pl.when(g < _NGROUP - 1)
            def _():
                fire(j0 + _NB + b, b)

        return carry

    lax.fori_loop(0, _NGROUP, group, 0)


def _transpose_body(in_ref, out_ref):
    x = in_ref[0]
    out_ref[0, :, : _D0 // 2] = x[:, : _OUTPUT_DIM].T
    out_ref[0, :, _D0 // 2 :] = x[:, _OUTPUT_DIM :].T


_transpose = pl.pallas_call(
    _transpose_body,
    grid=(_P,),
    in_specs=[pl.BlockSpec((1, _D0 // 2, 128), lambda i: (i, 0, 0))],
    out_specs=pl.BlockSpec((1, _OUTPUT_DIM, _D0), lambda i: (i, 0, 0)),
    out_shape=jax.ShapeDtypeStruct((_P, _OUTPUT_DIM, _D0), jnp.float32),
)


def kernel(inputs, pos_matrix):
    idx = (
        inputs.T.reshape(_P, 2, 32, _CHUNK // 2)
        .transpose(0, 2, 1, 3)
        .reshape(_NW, _NCHUNK, 2, _CHUNK // 2)
    )
    table = pos_matrix.reshape(_INPUT_DIM, -1)[:, :_OUTPUT_DIM]
    packed = _gather_kernel(idx, table).reshape(_P, _D0 // 2, 128)
    return _transpose(packed).transpose(2, 0, 1)
